# baseline (speedup 1.0000x reference)
import functools

import jax
import jax.numpy as jnp
from jax import lax
from jax.experimental import pallas as pl
from jax.experimental.pallas import tpu as pltpu
from jax.experimental.pallas import tpu_sc as plsc

_B = 16
_N = 2048
_HALF = _N // 2
_PTSW = _HALF * 3
_ITERS = _HALF // 16


def _sc_partials(planes, coef):
    mesh = plsc.VectorSubcoreMesh(core_axis_name="c", subcore_axis_name="s")

    @functools.partial(
        pl.kernel,
        out_type=jax.ShapeDtypeStruct((2, _B, 8, 16), jnp.float32),
        mesh=mesh,
        scratch_types=[
            pltpu.VMEM((14, _HALF), jnp.float32),
            pltpu.VMEM((36, 16), jnp.float32),
            pltpu.VMEM((8, 16), jnp.float32),
            pltpu.SemaphoreType.DMA,
        ],
        compiler_params=pltpu.CompilerParams(needs_layout_passes=False),
    )
    def k(planes_h, coef_h, out_h, sl_v, cf_v, out_v, sem):
        cid = lax.axis_index("c")
        sid = lax.axis_index("s")
        wid = sid * 2 + cid
        b = wid // 2
        h = wid % 2
        offw = h * _HALF

        cps = [
            pltpu.async_copy(planes_h.at[:, b, pl.ds(offw, _HALF)], sl_v, sem),
            pltpu.async_copy(coef_h.at[b], cf_v, sem),
        ]
        for cp in cps:
            cp.wait()

        zero = jnp.zeros((16,), jnp.float32)
        one = jnp.ones((16,), jnp.float32)
        cd = [cf_v[24 + j] for j in range(12)]

        def direction(base, crow, fbs0):
            ca = [cf_v[crow + j] for j in range(12)]

            def body(_, carry):
                wm, cnt, fbs, pos = carry
                sl = pl.ds(pos, 16)
                x = sl_v[base + 0, sl]
                y = sl_v[base + 1, sl]
                z = sl_v[base + 2, sl]
                fx = sl_v[base + 3, sl]
                fy = sl_v[base + 4, sl]
                fz = sl_v[base + 5, sl]
                w = sl_v[base + 6, sl]
                valid = (x == x) | (y == y) | (z == z)
                xc = jnp.where(valid, x, zero)
                yc = jnp.where(valid, y, zero)
                zc = jnp.where(valid, z, zero)
                fxc = jnp.where(valid, fx, zero)
                fyc = jnp.where(valid, fy, zero)
                fzc = jnp.where(valid, fz, zero)
                wc = jnp.where(valid, w, zero)
                e0 = ca[0] * xc + ca[1] * yc + ca[2] * zc + (ca[3] - fxc)
                e1 = ca[4] * xc + ca[5] * yc + ca[6] * zc + (ca[7] - fyc)
                e2 = ca[8] * xc + ca[9] * yc + ca[10] * zc + (ca[11] - fzc)
                q0 = cd[0] * xc + cd[1] * yc + cd[2] * zc + cd[3]
                q1 = cd[4] * xc + cd[5] * yc + cd[6] * zc + cd[7]
                q2 = cd[8] * xc + cd[9] * yc + cd[10] * zc + cd[11]
                wm = wm + (e0 * e0 + e1 * e1 + e2 * e2) * wc
                cnt = cnt + jnp.where(valid, one, zero)
                fbs = fbs + jnp.where(valid, q0 * q0 + q1 * q1 + q2 * q2, zero)
                return wm, cnt, fbs, pos + 16

            wm, cnt, fbs, _ = lax.fori_loop(
                0, _ITERS, body, (zero, zero, fbs0, jnp.int32(0)), unroll=4)
            return wm, cnt, fbs

        wm_f, cnt_f, fbs = direction(0, 0, zero)
        wm_b, cnt_b, fbs = direction(7, 12, fbs)

        out_v[0] = wm_f
        out_v[1] = cnt_f
        out_v[2] = wm_b
        out_v[3] = cnt_b
        out_v[4] = fbs
        out_v[5] = zero
        out_v[6] = zero
        out_v[7] = zero
        pltpu.sync_copy(out_v, out_h.at[h, b])

    return k(planes, coef)


def _combine_body(p_ref, o0_ref, o1_ref):
    x = p_ref[...]
    s = x[0] + x[1]
    wm_fw = jnp.sum(s[:, 0, :])
    cnt_fw = jnp.sum(s[:, 1, :])
    wm_bw = jnp.sum(s[:, 2, :])
    cnt_bw = jnp.sum(s[:, 3, :])
    loss0 = wm_fw / (3.0 * cnt_fw)
    loss1 = wm_bw / (3.0 * cnt_bw)
    o0_ref[...] = jnp.reshape(0.5 * (loss0 + loss1), (1, 1))
    fb_b = jnp.sum(s[:, 4, :], axis=1)
    cnt_b = jnp.sum(s[:, 1, :] + s[:, 3, :], axis=1)
    o1_ref[...] = jnp.reshape(jnp.mean(fb_b / cnt_b), (1, 1))


def _combine(parts):
    o0, o1 = pl.pallas_call(
        _combine_body,
        out_shape=[
            jax.ShapeDtypeStruct((1, 1), jnp.float32),
            jax.ShapeDtypeStruct((1, 1), jnp.float32),
        ],
    )(parts)
    return o0.reshape(()), o1.reshape(())


def kernel(pc0, static_flow_fw, static_aggr_trafo_fw, staticness_fw,
           pc1, static_flow_bw, static_aggr_trafo_bw, staticness_bw):
    tf_fw = jax.lax.stop_gradient(static_aggr_trafo_fw).astype(jnp.float32)
    tf_bw = jax.lax.stop_gradient(static_aggr_trafo_bw).astype(jnp.float32)
    dt = tf_fw.dtype
    eye3 = jnp.eye(3, dtype=dt)
    a_fw = jnp.concatenate(
        [tf_fw[:, :3, :3] - eye3, tf_fw[:, :3, 3:4]], axis=2).reshape(_B, 12)
    a_bw = jnp.concatenate(
        [tf_bw[:, :3, :3] - eye3, tf_bw[:, :3, 3:4]], axis=2).reshape(_B, 12)
    fb = jnp.sum(tf_bw[:, :, :, None] * tf_fw[:, None, :, :], axis=2)
    d = (fb[:, :3, :] - jnp.eye(4, dtype=dt)[None, :3, :]).reshape(_B, 12)
    coef = jnp.concatenate([a_fw, a_bw, d], axis=1).astype(jnp.float32)
    coef16 = jnp.broadcast_to(coef[:, :, None], (_B, 36, 16))

    planes = jnp.stack(
        [pc0[:, :, 0], pc0[:, :, 1], pc0[:, :, 2],
         static_flow_fw[:, :, 0], static_flow_fw[:, :, 1],
         static_flow_fw[:, :, 2], staticness_fw,
         pc1[:, :, 0], pc1[:, :, 1], pc1[:, :, 2],
         static_flow_bw[:, :, 0], static_flow_bw[:, :, 1],
         static_flow_bw[:, :, 2], staticness_bw], axis=0)
    parts = _sc_partials(planes, coef16)
    return _combine(parts)


# speedup vs baseline: 1.0243x; 1.0243x over previous
import functools

import jax
import jax.numpy as jnp
from jax import lax
from jax.experimental import pallas as pl
from jax.experimental.pallas import tpu as pltpu
from jax.experimental.pallas import tpu_sc as plsc

_B = 16
_N = 2048
_HALF = _N // 2
_PTSW = _HALF * 3
_ITERS = _HALF // 16


def _sc_partials(planes, coef):
    mesh = plsc.VectorSubcoreMesh(core_axis_name="c", subcore_axis_name="s")

    @functools.partial(
        pl.kernel,
        out_type=jax.ShapeDtypeStruct((2, _B, 8, 16), jnp.float32),
        mesh=mesh,
        scratch_types=[
            pltpu.VMEM((14, _HALF), jnp.float32),
            pltpu.VMEM((40,), jnp.float32),
            pltpu.VMEM((8, 16), jnp.float32),
            pltpu.SemaphoreType.DMA,
        ],
        compiler_params=pltpu.CompilerParams(needs_layout_passes=False),
    )
    def k(planes_h, coef_h, out_h, sl_v, cf_v, out_v, sem):
        cid = lax.axis_index("c")
        sid = lax.axis_index("s")
        wid = sid * 2 + cid
        b = wid // 2
        h = wid % 2
        offw = h * _HALF

        cps = [
            pltpu.async_copy(planes_h.at[:, b, pl.ds(offw, _HALF)], sl_v, sem),
            pltpu.async_copy(coef_h.at[b], cf_v, sem),
        ]
        for cp in cps:
            cp.wait()

        zero = jnp.zeros((16,), jnp.float32)
        one = jnp.ones((16,), jnp.float32)

        def bc(j):
            return plsc.load_gather(cf_v, [jnp.full((16,), j, jnp.int32)])

        cd = [bc(24 + j) for j in range(12)]

        def direction(base, crow, fbs0):
            ca = [bc(crow + j) for j in range(12)]

            def body(_, carry):
                wm, cnt, fbs, pos = carry
                sl = pl.ds(pos, 16)
                x = sl_v[base + 0, sl]
                y = sl_v[base + 1, sl]
                z = sl_v[base + 2, sl]
                fx = sl_v[base + 3, sl]
                fy = sl_v[base + 4, sl]
                fz = sl_v[base + 5, sl]
                w = sl_v[base + 6, sl]
                valid = (x == x) | (y == y) | (z == z)
                xc = jnp.where(valid, x, zero)
                yc = jnp.where(valid, y, zero)
                zc = jnp.where(valid, z, zero)
                fxc = jnp.where(valid, fx, zero)
                fyc = jnp.where(valid, fy, zero)
                fzc = jnp.where(valid, fz, zero)
                wc = jnp.where(valid, w, zero)
                e0 = ca[0] * xc + ca[1] * yc + ca[2] * zc + (ca[3] - fxc)
                e1 = ca[4] * xc + ca[5] * yc + ca[6] * zc + (ca[7] - fyc)
                e2 = ca[8] * xc + ca[9] * yc + ca[10] * zc + (ca[11] - fzc)
                q0 = cd[0] * xc + cd[1] * yc + cd[2] * zc + cd[3]
                q1 = cd[4] * xc + cd[5] * yc + cd[6] * zc + cd[7]
                q2 = cd[8] * xc + cd[9] * yc + cd[10] * zc + cd[11]
                wm = wm + (e0 * e0 + e1 * e1 + e2 * e2) * wc
                cnt = cnt + jnp.where(valid, one, zero)
                fbs = fbs + jnp.where(valid, q0 * q0 + q1 * q1 + q2 * q2, zero)
                return wm, cnt, fbs, pos + 16

            wm, cnt, fbs, _ = lax.fori_loop(
                0, _ITERS, body, (zero, zero, fbs0, jnp.int32(0)))
            return wm, cnt, fbs

        wm_f, cnt_f, fbs = direction(0, 0, zero)
        wm_b, cnt_b, fbs = direction(7, 12, fbs)

        out_v[0] = wm_f
        out_v[1] = cnt_f
        out_v[2] = wm_b
        out_v[3] = cnt_b
        out_v[4] = fbs
        out_v[5] = zero
        out_v[6] = zero
        out_v[7] = zero
        pltpu.sync_copy(out_v, out_h.at[h, b])

    return k(planes, coef)


def _combine_body(p_ref, o0_ref, o1_ref):
    x = p_ref[...]
    s = x[0] + x[1]
    wm_fw = jnp.sum(s[:, 0, :])
    cnt_fw = jnp.sum(s[:, 1, :])
    wm_bw = jnp.sum(s[:, 2, :])
    cnt_bw = jnp.sum(s[:, 3, :])
    loss0 = wm_fw / (3.0 * cnt_fw)
    loss1 = wm_bw / (3.0 * cnt_bw)
    o0_ref[...] = jnp.reshape(0.5 * (loss0 + loss1), (1, 1))
    fb_b = jnp.sum(s[:, 4, :], axis=1)
    cnt_b = jnp.sum(s[:, 1, :] + s[:, 3, :], axis=1)
    o1_ref[...] = jnp.reshape(jnp.mean(fb_b / cnt_b), (1, 1))


def _combine(parts):
    o0, o1 = pl.pallas_call(
        _combine_body,
        out_shape=[
            jax.ShapeDtypeStruct((1, 1), jnp.float32),
            jax.ShapeDtypeStruct((1, 1), jnp.float32),
        ],
    )(parts)
    return o0.reshape(()), o1.reshape(())


def kernel(pc0, static_flow_fw, static_aggr_trafo_fw, staticness_fw,
           pc1, static_flow_bw, static_aggr_trafo_bw, staticness_bw):
    tf_fw = jax.lax.stop_gradient(static_aggr_trafo_fw).astype(jnp.float32)
    tf_bw = jax.lax.stop_gradient(static_aggr_trafo_bw).astype(jnp.float32)
    dt = tf_fw.dtype
    eye3 = jnp.eye(3, dtype=dt)
    a_fw = jnp.concatenate(
        [tf_fw[:, :3, :3] - eye3, tf_fw[:, :3, 3:4]], axis=2).reshape(_B, 12)
    a_bw = jnp.concatenate(
        [tf_bw[:, :3, :3] - eye3, tf_bw[:, :3, 3:4]], axis=2).reshape(_B, 12)
    fb = jnp.sum(tf_bw[:, :, :, None] * tf_fw[:, None, :, :], axis=2)
    d = (fb[:, :3, :] - jnp.eye(4, dtype=dt)[None, :3, :]).reshape(_B, 12)
    coef = jnp.concatenate(
        [a_fw, a_bw, d, jnp.zeros((_B, 4), dt)], axis=1).astype(jnp.float32)

    planes = jnp.stack(
        [pc0[:, :, 0], pc0[:, :, 1], pc0[:, :, 2],
         static_flow_fw[:, :, 0], static_flow_fw[:, :, 1],
         static_flow_fw[:, :, 2], staticness_fw,
         pc1[:, :, 0], pc1[:, :, 1], pc1[:, :, 2],
         static_flow_bw[:, :, 0], static_flow_bw[:, :, 1],
         static_flow_bw[:, :, 2], staticness_bw], axis=0)
    parts = _sc_partials(planes, coef)
    return _combine(parts)
